# Initial kernel scaffold; baseline (speedup 1.0000x reference)
import jax
import jax.numpy as jnp
from jax.experimental import pallas as pl


def kernel(x, edge_index, edge_attr, W1, att_s1, att_d1, We1, att_e1, b1, W2, att_s2, att_d2, We2, att_e2, b2, Wfc, bfc):
    raise NotImplementedError("write your pallas kernel here")


# speedup vs baseline: 2.6538x; 2.6538x over previous
import functools
import jax
import jax.numpy as jnp
from jax.experimental import pallas as pl
from jax.experimental.pallas import tpu as pltpu

BLK = 128
EROW = 128


def _mm_body(x_ref, w_ref, b_ref, o_ref, *, relu):
    acc = jnp.dot(x_ref[...], w_ref[...], preferred_element_type=jnp.float32)
    acc = acc + b_ref[...]
    if relu:
        acc = jnp.maximum(acc, 0.0)
    o_ref[...] = acc


def _matmul(x, w, b, relu=False, block_rows=1024):
    m, k = x.shape
    n = w.shape[1]
    grid = m // block_rows
    return pl.pallas_call(
        functools.partial(_mm_body, relu=relu),
        grid=(grid,),
        in_specs=[
            pl.BlockSpec((block_rows, k), lambda i: (i, 0)),
            pl.BlockSpec((k, n), lambda i: (0, 0)),
            pl.BlockSpec((1, n), lambda i: (0, 0)),
        ],
        out_specs=pl.BlockSpec((block_rows, n), lambda i: (i, 0)),
        out_shape=jax.ShapeDtypeStruct((m, n), jnp.float32),
    )(x, w, b)


def _mean_body(x_ref, o_ref, *, count):
    o_ref[...] = jnp.sum(x_ref[...], keepdims=True) / count


def _mean_scalar(flat2d, count):
    return pl.pallas_call(
        functools.partial(_mean_body, count=count),
        out_shape=jax.ShapeDtypeStruct((1, 1), jnp.float32),
    )(flat2d)[0, 0]


def _gat_body(starts_ref, hg_ref, adst_ref, dst_ref, ea_ref, we_ref, atte_ref,
              bias_ref, o_ref, gbuf, sem, *, relu, hid, haug):
    b = pl.program_id(0)
    c = jnp.sum(we_ref[...] * atte_ref[...])
    adst_vec = adst_ref[0, 0]
    rs = starts_ref[b] // EROW
    re = (starts_ref[b + 1] + (EROW - 1)) // EROW
    lane_ids = jax.lax.broadcasted_iota(jnp.int32, (EROW, BLK), 1)

    def body(j, carry):
        msg, den = carry
        cp = pltpu.make_async_copy(
            hg_ref.at[pl.ds(j * EROW, EROW)], gbuf, sem)
        cp.start()
        dsts = dst_ref[pl.ds(j, 1), :][0]
        eas = ea_ref[pl.ds(j, 1), :][0]
        dstl = dsts - b * BLK
        oh = (dstl[:, None] == lane_ids).astype(jnp.float32)
        a_d = jnp.dot(oh, adst_vec, preferred_element_type=jnp.float32)
        cp.wait()
        g = gbuf[...]
        a_s = g[:, hid]
        alpha = a_s + a_d + eas * c
        alpha = jnp.where(alpha >= 0.0, alpha, 0.2 * alpha)
        ex = jnp.exp(alpha)
        msg = msg + jnp.dot(oh.T, (g * ex[:, None])[:, :hid],
                            preferred_element_type=jnp.float32)
        den = den + jnp.dot(oh.T, ex[:, None],
                            preferred_element_type=jnp.float32)
        return msg, den

    msg0 = jnp.zeros((BLK, hid), jnp.float32)
    den0 = jnp.zeros((BLK, 1), jnp.float32)
    msg, den = jax.lax.fori_loop(rs, re, body, (msg0, den0))
    out = msg / (den + 1e-16) + bias_ref[...]
    if relu:
        out = jnp.maximum(out, 0.0)
    o_ref[...] = out


def _gat_layer(hg, adst_t, dsts2d, eas2d, starts, We, att_e, bias, relu):
    haug = hg.shape[1]
    hid = haug - 128
    nblk = adst_t.shape[0]
    np_ = nblk * BLK
    return pl.pallas_call(
        functools.partial(_gat_body, relu=relu, hid=hid, haug=haug),
        grid=(nblk,),
        in_specs=[
            pl.BlockSpec(memory_space=pltpu.SMEM),
            pl.BlockSpec(memory_space=pl.ANY),
            pl.BlockSpec((1, 1, BLK), lambda i: (i, 0, 0)),
            pl.BlockSpec(dsts2d.shape, lambda i: (0, 0)),
            pl.BlockSpec(eas2d.shape, lambda i: (0, 0)),
            pl.BlockSpec((1, We.shape[1]), lambda i: (0, 0)),
            pl.BlockSpec((1, att_e.shape[1]), lambda i: (0, 0)),
            pl.BlockSpec((1, hid), lambda i: (0, 0)),
        ],
        out_specs=pl.BlockSpec((BLK, hid), lambda i: (i, 0)),
        out_shape=jax.ShapeDtypeStruct((np_, hid), jnp.float32),
        scratch_shapes=[
            pltpu.VMEM((EROW, haug), jnp.float32),
            pltpu.SemaphoreType.DMA,
        ],
    )(starts, hg, adst_t, dsts2d, eas2d, We, att_e, bias)


def _augment(W, att_s, att_d):
    hid = W.shape[1]
    ws = (W @ att_s)[:, None]
    wd = (W @ att_d)[:, None]
    pad = jnp.zeros((W.shape[0], 128 - 2), W.dtype)
    return jnp.concatenate([W, ws, wd, pad], axis=1)


def kernel(x, edge_index, edge_attr, W1, att_s1, att_d1, We1, att_e1, b1,
           W2, att_s2, att_d2, We2, att_e2, b2, Wfc, bfc):
    n, in_dim = x.shape
    e = edge_index.shape[1]
    np_ = ((n + 1279) // 1280) * 1280
    x_pad = jnp.pad(x, ((0, np_ - n), (0, 0)))

    ei = edge_index.astype(jnp.int32)
    loop = jnp.arange(n, dtype=jnp.int32)
    ea_flat = edge_attr[:, 0]
    erows = e // EROW
    mean_ea = _mean_scalar(ea_flat.reshape(erows, EROW), float(e))
    src_all = jnp.concatenate([ei[0], loop])
    dst_all = jnp.concatenate([ei[1], loop])
    ea_all = jnp.concatenate([ea_flat, jnp.full((n,), mean_ea, jnp.float32)])
    tot = e + n
    tot_pad = ((tot + EROW - 1) // EROW) * EROW
    padn = tot_pad - tot
    src_all = jnp.pad(src_all, (0, padn))
    dst_all = jnp.pad(dst_all, (0, padn), constant_values=jnp.int32(2**30))
    ea_all = jnp.pad(ea_all, (0, padn))
    perm = jnp.argsort(dst_all)
    srcs = src_all[perm]
    dsts = dst_all[perm]
    eas = ea_all[perm]
    nblk = np_ // BLK
    starts = jnp.searchsorted(
        dsts, jnp.arange(nblk + 1, dtype=jnp.int32) * BLK).astype(jnp.int32)
    srcs2d = srcs.reshape(-1, EROW)
    dsts2d = dsts.reshape(-1, EROW)
    eas2d = eas.reshape(-1, EROW)

    zeros640 = jnp.zeros((1, W1.shape[1] + 128), jnp.float32)

    W1a = _augment(W1, att_s1, att_d1)
    h1_aug = _matmul(x_pad, W1a, zeros640)
    hg1 = h1_aug[srcs]
    adst1 = h1_aug[:, W1.shape[1] + 1].reshape(-1, 1, BLK)
    g1 = _gat_layer(hg1, adst1, dsts2d, eas2d, starts,
                    We1.reshape(1, -1), att_e1.reshape(1, -1),
                    b1.reshape(1, -1), relu=True)

    W2a = _augment(W2, att_s2, att_d2)
    h2_aug = _matmul(g1, W2a, zeros640)
    hg2 = h2_aug[srcs]
    adst2 = h2_aug[:, W2.shape[1] + 1].reshape(-1, 1, BLK)
    g2 = _gat_layer(hg2, adst2, dsts2d, eas2d, starts,
                    We2.reshape(1, -1), att_e2.reshape(1, -1),
                    b2.reshape(1, -1), relu=False)

    out = _matmul(g2, Wfc, bfc.reshape(1, -1))
    return out[:n]
